# baseline (speedup 1.0000x reference)
import functools

import numpy as np
import jax
import jax.numpy as jnp
from jax import lax
from jax.experimental import pallas as pl
from jax.experimental.pallas import tpu as pltpu
from jax.experimental.pallas import tpu_sc as plsc

D_MODEL = 768
HEADS = 12
DH = 64
D_FF = 512
S = 4096
U = 45
UP = 48
BR = 512

NTILES = 32
CHUNK = 512
NCH = S // CHUNK
NTASKS = HEADS * NCH
TPW = NTASKS // NTILES
DSL = 8
NPASS = DH // DSL

_INTERPRET = False

_rng = np.random.default_rng(0)
_IDX = _rng.integers(0, S, size=(S, U)).astype(np.int32)
_IDXT = np.ascontiguousarray(
    _IDX.T.reshape(45, S // 512, 512).transpose(1, 0, 2).reshape(S // 512, -1))


def _proj_body(x_ref, wq_ref, wk_ref, wv_ref, q_ref, v_ref, qt_ref, kt_ref):
    x = x_ref[...]
    yq = jnp.dot(x, wq_ref[...], preferred_element_type=jnp.float32)
    yk = jnp.dot(x, wk_ref[...], preferred_element_type=jnp.float32)
    yv = jnp.dot(x, wv_ref[...], preferred_element_type=jnp.float32)
    for h in range(HEADS):
        sq = yq[:, h * DH:(h + 1) * DH]
        q_ref[h] = sq
        v_ref[h] = yv[:, h * DH:(h + 1) * DH]
        qt_ref[0, h] = sq.T
        kt_ref[h] = yk[:, h * DH:(h + 1) * DH].T


def _sc_measure(qt_hbm, kt_hbm, idxt_hbm, m_hbm,
                kt_buf, qt_buf, idx_buf, p_buf, m_buf):
    wid = lax.axis_index("s") * 2 + lax.axis_index("c")

    def task_body(j, carry0):
        task = wid * TPW + j
        head = task // NCH
        chunk = task % NCH
        q0 = chunk * CHUNK
        pltpu.sync_copy(idxt_hbm.at[chunk], idx_buf)
        pltpu.sync_copy(qt_hbm.at[chunk, head], qt_buf)

        def zero_body(g, carry):
            p_buf[pl.ds(g * 16, 16)] = jnp.zeros((16,), jnp.float32)
            return carry

        lax.fori_loop(0, CHUNK // 16 * U, zero_body, 0)

        def pass_body(dsl, carry):
            pltpu.sync_copy(
                kt_hbm.at[head, pl.ds(dsl * (DSL * S), DSL * S)], kt_buf)

            def pair_body(p, carry2):
                for grp in range(2):
                    lanes0 = p * 32 + grp * 16
                    qv = [qt_buf[pl.ds(dsl * (DSL * CHUNK) + dd * CHUNK
                                       + lanes0, 16)]
                          for dd in range(DSL)]
                    gl = p * 2 + grp
                    for u in range(U):
                        keys = idx_buf[pl.ds(u * CHUNK + lanes0, 16)]
                        pr = qv[0] * plsc.load_gather(kt_buf, [keys])
                        for dd in range(1, DSL):
                            g = plsc.load_gather(
                                kt_buf, [keys + jnp.int32(dd * S)])
                            pr = pr + g * qv[dd]
                        row = (gl * U + u) * 16
                        p_buf[pl.ds(row, 16)] = p_buf[pl.ds(row, 16)] + pr
                return carry2

            lax.fori_loop(0, CHUNK // 32, pair_body, 0)
            return carry

        lax.fori_loop(0, NPASS, pass_body, 0)

        def fin_body(g, carry):
            mmax = p_buf[pl.ds(g * U * 16, 16)]
            msum = p_buf[pl.ds(g * U * 16, 16)]
            for u in range(1, U):
                v = p_buf[pl.ds((g * U + u) * 16, 16)]
                mmax = jnp.maximum(mmax, v)
                msum = msum + v
            m_buf[pl.ds(g * 16, 16)] = mmax - msum * (1.0 / U)
            return carry

        lax.fori_loop(0, CHUNK // 16, fin_body, 0)
        pltpu.sync_copy(m_buf, m_hbm.at[head, pl.ds(q0, CHUNK)])
        return carry0

    lax.fori_loop(0, TPW, task_body, 0)


def _topk_body(m_ref, top_ref, scr):
    scr[...] = m_ref[0]
    iota = jax.lax.broadcasted_iota(jnp.int32, (1, S), 1)
    lane64 = jax.lax.broadcasted_iota(jnp.int32, (1, 64), 1)

    def step(u, acc):
        row = scr[...]
        m = jnp.max(row)
        idx = jnp.min(jnp.where(row == m, iota, jnp.int32(2**30)))
        scr[...] = jnp.where(iota == idx, -jnp.inf, row)
        return jnp.where(lane64 == u, idx, acc)

    top_ref[0] = jax.lax.fori_loop(0, U, step, jnp.zeros((1, 64), jnp.int32))


def _attn_body(top_smem, q_ref, kt_ref, v_ref, out_ref, qsel):
    h = pl.program_id(0)
    for u in range(U):
        i = top_smem[h * 64 + u]
        qsel[pl.ds(u, 1), :] = q_ref[0, pl.ds(i, 1), :]
    scores = jax.lax.dot_general(
        qsel[...], kt_ref[0], (((1,), (0,)), ((), ())),
        preferred_element_type=jnp.float32) * (1.0 / 8.0)
    smax = jnp.max(scores, axis=1, keepdims=True)
    e = jnp.exp(scores - smax)
    att = e / jnp.sum(e, axis=1, keepdims=True)
    ctx = jnp.dot(att, v_ref[0], preferred_element_type=jnp.float32)
    vmean = jnp.mean(v_ref[0], axis=0, keepdims=True)
    out_ref[0] = jnp.broadcast_to(vmean, (S, DH))
    for u in range(U):
        i = top_smem[h * 64 + u]
        out_ref[0, pl.ds(i, 1), :] = ctx[u:u + 1, :]


def _ln(y, g, b):
    mu = jnp.mean(y, axis=-1, keepdims=True)
    var = jnp.mean((y - mu) ** 2, axis=-1, keepdims=True)
    return (y - mu) / jnp.sqrt(var + 1e-3) * g + b


def _epilogue_body(ctx_ref, x_ref, wo_ref, wff1_ref, bff1_ref, wff2_ref,
                   bff2_ref, g1_ref, b1_ref, g2_ref, b2_ref, out_ref):
    ctx = jnp.concatenate([ctx_ref[h] for h in range(HEADS)], axis=1)
    attn = jnp.dot(ctx, wo_ref[...], preferred_element_type=jnp.float32)
    h1 = _ln(x_ref[...] + attn, g1_ref[...], b1_ref[...])
    ffa = jnp.maximum(
        jnp.dot(h1, wff1_ref[...], preferred_element_type=jnp.float32)
        + bff1_ref[...], 0.0)
    ff = jnp.dot(ffa, wff2_ref[...], preferred_element_type=jnp.float32) + bff2_ref[...]
    out_ref[...] = _ln(h1 + ff, g2_ref[...], b2_ref[...])


def kernel(x, Wq, Wk, Wv, Wo, Wff1, bff1, Wff2, bff2, ln1_g, ln1_b, ln2_g, ln2_b):
    B = x.shape[0]
    x2 = x.reshape(S, D_MODEL)

    q, v, qt, kt = pl.pallas_call(
        _proj_body,
        grid=(S // BR,),
        in_specs=[
            pl.BlockSpec((BR, D_MODEL), lambda i: (i, 0)),
            pl.BlockSpec((D_MODEL, D_MODEL), lambda i: (0, 0)),
            pl.BlockSpec((D_MODEL, D_MODEL), lambda i: (0, 0)),
            pl.BlockSpec((D_MODEL, D_MODEL), lambda i: (0, 0)),
        ],
        out_specs=[
            pl.BlockSpec((HEADS, BR, DH), lambda i: (0, i, 0)),
            pl.BlockSpec((HEADS, BR, DH), lambda i: (0, i, 0)),
            pl.BlockSpec((1, HEADS, DH, BR), lambda i: (i, 0, 0, 0)),
            pl.BlockSpec((HEADS, DH, BR), lambda i: (0, 0, i)),
        ],
        out_shape=[
            jax.ShapeDtypeStruct((HEADS, S, DH), jnp.float32),
            jax.ShapeDtypeStruct((HEADS, S, DH), jnp.float32),
            jax.ShapeDtypeStruct((S // BR, HEADS, DH, BR), jnp.float32),
            jax.ShapeDtypeStruct((HEADS, DH, S), jnp.float32),
        ],
        interpret=_INTERPRET,
    )(x2, Wq, Wk, Wv)

    idxt = jnp.asarray(_IDXT)
    m = pl.kernel(
        _sc_measure,
        out_type=jax.ShapeDtypeStruct((HEADS, S), jnp.float32),
        mesh=plsc.VectorSubcoreMesh(core_axis_name="c", subcore_axis_name="s"),
        compiler_params=pltpu.CompilerParams(needs_layout_passes=False),
        scratch_types=[
            pltpu.VMEM((DSL * S,), jnp.float32),
            pltpu.VMEM((DH * CHUNK,), jnp.float32),
            pltpu.VMEM((U * CHUNK,), jnp.int32),
            pltpu.VMEM((CHUNK // 16 * U * 16,), jnp.float32),
            pltpu.VMEM((CHUNK,), jnp.float32),
        ],
    )(qt.reshape(NCH, HEADS, DH * CHUNK), kt.reshape(HEADS, DH * S), idxt)

    m_top = pl.pallas_call(
        _topk_body,
        grid=(HEADS,),
        in_specs=[pl.BlockSpec((1, 1, S), lambda h: (h, 0, 0))],
        out_specs=pl.BlockSpec((1, 1, 64), lambda h: (h, 0, 0)),
        out_shape=jax.ShapeDtypeStruct((HEADS, 1, 64), jnp.int32),
        scratch_shapes=[pltpu.VMEM((1, S), jnp.float32)],
        interpret=_INTERPRET,
    )(m.reshape(HEADS, 1, S))

    ctx = pl.pallas_call(
        _attn_body,
        grid_spec=pltpu.PrefetchScalarGridSpec(
            num_scalar_prefetch=1,
            grid=(HEADS,),
            in_specs=[
                pl.BlockSpec((1, S, DH), lambda h, *_: (h, 0, 0)),
                pl.BlockSpec((1, DH, S), lambda h, *_: (h, 0, 0)),
                pl.BlockSpec((1, S, DH), lambda h, *_: (h, 0, 0)),
            ],
            out_specs=pl.BlockSpec((1, S, DH), lambda h, *_: (h, 0, 0)),
            scratch_shapes=[pltpu.VMEM((UP, DH), jnp.float32)],
        ),
        out_shape=jax.ShapeDtypeStruct((HEADS, S, DH), jnp.float32),
        interpret=_INTERPRET,
    )(m_top.reshape(-1), q, kt, v)

    out = pl.pallas_call(
        _epilogue_body,
        grid=(S // BR,),
        in_specs=[
            pl.BlockSpec((HEADS, BR, DH), lambda i: (0, i, 0)),
            pl.BlockSpec((BR, D_MODEL), lambda i: (i, 0)),
            pl.BlockSpec((D_MODEL, D_MODEL), lambda i: (0, 0)),
            pl.BlockSpec((D_MODEL, D_FF), lambda i: (0, 0)),
            pl.BlockSpec((1, D_FF), lambda i: (0, 0)),
            pl.BlockSpec((D_FF, D_MODEL), lambda i: (0, 0)),
            pl.BlockSpec((1, D_MODEL), lambda i: (0, 0)),
            pl.BlockSpec((1, D_MODEL), lambda i: (0, 0)),
            pl.BlockSpec((1, D_MODEL), lambda i: (0, 0)),
            pl.BlockSpec((1, D_MODEL), lambda i: (0, 0)),
            pl.BlockSpec((1, D_MODEL), lambda i: (0, 0)),
        ],
        out_specs=pl.BlockSpec((BR, D_MODEL), lambda i: (i, 0)),
        out_shape=jax.ShapeDtypeStruct((S, D_MODEL), jnp.float32),
        interpret=_INTERPRET,
    )(ctx, x2, Wo, Wff1, bff1.reshape(1, -1), Wff2, bff2.reshape(1, -1),
      ln1_g.reshape(1, -1), ln1_b.reshape(1, -1),
      ln2_g.reshape(1, -1), ln2_b.reshape(1, -1))

    return out.reshape(B, S, D_MODEL)


# speedup vs baseline: 4.0770x; 4.0770x over previous
import functools

import numpy as np
import jax
import jax.numpy as jnp
from jax import lax
from jax.experimental import pallas as pl
from jax.experimental.pallas import tpu as pltpu
from jax.experimental.pallas import tpu_sc as plsc

D_MODEL = 768
HEADS = 12
DH = 64
D_FF = 512
S = 4096
U = 45
UP = 48
BR = 512

NTILES = 32
CHUNK = 512
NCH = S // CHUNK
NTASKS = HEADS * NCH
TPW = NTASKS // NTILES
DSL = 8
NPASS = DH // DSL

_INTERPRET = False

_rng = np.random.default_rng(0)
_IDX = _rng.integers(0, S, size=(S, U)).astype(np.int32)
_IDXT = np.ascontiguousarray(
    _IDX.T.reshape(45, S // 512, 512).transpose(1, 0, 2).reshape(S // 512, -1))


def _proj_body(x_ref, wq_ref, wk_ref, wv_ref, q_ref, v_ref, qt_ref, kt_ref):
    x = x_ref[...]
    yq = jnp.dot(x, wq_ref[...], preferred_element_type=jnp.float32)
    yk = jnp.dot(x, wk_ref[...], preferred_element_type=jnp.float32)
    yv = jnp.dot(x, wv_ref[...], preferred_element_type=jnp.float32)
    for h in range(HEADS):
        sq = yq[:, h * DH:(h + 1) * DH]
        q_ref[h] = sq
        v_ref[h] = yv[:, h * DH:(h + 1) * DH]
        qt_ref[0, h] = sq.T
        kt_ref[h] = yk[:, h * DH:(h + 1) * DH].T


def _sc_measure(qt_hbm, kt_hbm, idxt_hbm, m_hbm,
                kt_buf, qt_buf, idx_buf, p_buf, m_buf):
    wid = lax.axis_index("s") * 2 + lax.axis_index("c")

    def task_body(j, carry0):
        task = wid * TPW + j
        head = task // NCH
        chunk = task % NCH
        q0 = chunk * CHUNK
        pltpu.sync_copy(idxt_hbm.at[chunk], idx_buf)
        pltpu.sync_copy(qt_hbm.at[chunk, head], qt_buf)

        def zero_body(g, carry):
            p_buf[pl.ds(g * 16, 16)] = jnp.zeros((16,), jnp.float32)
            return carry

        lax.fori_loop(0, CHUNK // 16 * U, zero_body, 0)

        def pass_body(dsl, carry):
            pltpu.sync_copy(
                kt_hbm.at[head, pl.ds(dsl * (DSL * S), DSL * S)], kt_buf)

            def pair_body(p, carry2):
                for grp in range(2):
                    lanes0 = p * 32 + grp * 16
                    qv = [qt_buf[pl.ds(dsl * (DSL * CHUNK) + dd * CHUNK
                                       + lanes0, 16)]
                          for dd in range(DSL)]
                    gl = p * 2 + grp
                    @functools.partial(plsc.parallel_loop, 0, U, unroll=5)
                    def _u_body(u):
                        keys = idx_buf[pl.ds(u * CHUNK + lanes0, 16)]
                        row = (gl * U + u) * 16
                        pold = p_buf[pl.ds(row, 16)]
                        gs = [plsc.load_gather(kt_buf, [keys])]
                        for dd in range(1, DSL):
                            gs.append(plsc.load_gather(
                                kt_buf, [keys + jnp.int32(dd * S)]))
                        ps = [g * qq for g, qq in zip(gs, qv)]
                        while len(ps) > 1:
                            ps = [ps[i] + ps[i + 1]
                                  for i in range(0, len(ps), 2)]
                        p_buf[pl.ds(row, 16)] = pold + ps[0]
                return carry2

            lax.fori_loop(0, CHUNK // 32, pair_body, 0)
            return carry

        lax.fori_loop(0, NPASS, pass_body, 0)

        def fin_body(g, carry):
            mmax = p_buf[pl.ds(g * U * 16, 16)]
            msum = p_buf[pl.ds(g * U * 16, 16)]
            for u in range(1, U):
                v = p_buf[pl.ds((g * U + u) * 16, 16)]
                mmax = jnp.maximum(mmax, v)
                msum = msum + v
            m_buf[pl.ds(g * 16, 16)] = mmax - msum * (1.0 / U)
            return carry

        lax.fori_loop(0, CHUNK // 16, fin_body, 0)
        pltpu.sync_copy(m_buf, m_hbm.at[head, pl.ds(q0, CHUNK)])
        return carry0

    lax.fori_loop(0, TPW, task_body, 0)


def _topk_body(m_ref, top_ref, scr):
    scr[...] = m_ref[0]
    iota = jax.lax.broadcasted_iota(jnp.int32, (1, S), 1)
    lane64 = jax.lax.broadcasted_iota(jnp.int32, (1, 64), 1)

    def step(u, acc):
        row = scr[...]
        m = jnp.max(row)
        idx = jnp.min(jnp.where(row == m, iota, jnp.int32(2**30)))
        scr[...] = jnp.where(iota == idx, -jnp.inf, row)
        return jnp.where(lane64 == u, idx, acc)

    top_ref[0] = jax.lax.fori_loop(0, U, step, jnp.zeros((1, 64), jnp.int32))


def _attn_body(top_smem, q_ref, kt_ref, v_ref, out_ref, qsel):
    h = pl.program_id(0)
    for u in range(U):
        i = top_smem[h * 64 + u]
        qsel[pl.ds(u, 1), :] = q_ref[0, pl.ds(i, 1), :]
    scores = jax.lax.dot_general(
        qsel[...], kt_ref[0], (((1,), (0,)), ((), ())),
        preferred_element_type=jnp.float32) * (1.0 / 8.0)
    smax = jnp.max(scores, axis=1, keepdims=True)
    e = jnp.exp(scores - smax)
    att = e / jnp.sum(e, axis=1, keepdims=True)
    ctx = jnp.dot(att, v_ref[0], preferred_element_type=jnp.float32)
    vmean = jnp.mean(v_ref[0], axis=0, keepdims=True)
    out_ref[0] = jnp.broadcast_to(vmean, (S, DH))
    for u in range(U):
        i = top_smem[h * 64 + u]
        out_ref[0, pl.ds(i, 1), :] = ctx[u:u + 1, :]


def _ln(y, g, b):
    mu = jnp.mean(y, axis=-1, keepdims=True)
    var = jnp.mean((y - mu) ** 2, axis=-1, keepdims=True)
    return (y - mu) / jnp.sqrt(var + 1e-3) * g + b


def _epilogue_body(ctx_ref, x_ref, wo_ref, wff1_ref, bff1_ref, wff2_ref,
                   bff2_ref, g1_ref, b1_ref, g2_ref, b2_ref, out_ref):
    ctx = jnp.concatenate([ctx_ref[h] for h in range(HEADS)], axis=1)
    attn = jnp.dot(ctx, wo_ref[...], preferred_element_type=jnp.float32)
    h1 = _ln(x_ref[...] + attn, g1_ref[...], b1_ref[...])
    ffa = jnp.maximum(
        jnp.dot(h1, wff1_ref[...], preferred_element_type=jnp.float32)
        + bff1_ref[...], 0.0)
    ff = jnp.dot(ffa, wff2_ref[...], preferred_element_type=jnp.float32) + bff2_ref[...]
    out_ref[...] = _ln(h1 + ff, g2_ref[...], b2_ref[...])


def kernel(x, Wq, Wk, Wv, Wo, Wff1, bff1, Wff2, bff2, ln1_g, ln1_b, ln2_g, ln2_b):
    B = x.shape[0]
    x2 = x.reshape(S, D_MODEL)

    q, v, qt, kt = pl.pallas_call(
        _proj_body,
        grid=(S // BR,),
        in_specs=[
            pl.BlockSpec((BR, D_MODEL), lambda i: (i, 0)),
            pl.BlockSpec((D_MODEL, D_MODEL), lambda i: (0, 0)),
            pl.BlockSpec((D_MODEL, D_MODEL), lambda i: (0, 0)),
            pl.BlockSpec((D_MODEL, D_MODEL), lambda i: (0, 0)),
        ],
        out_specs=[
            pl.BlockSpec((HEADS, BR, DH), lambda i: (0, i, 0)),
            pl.BlockSpec((HEADS, BR, DH), lambda i: (0, i, 0)),
            pl.BlockSpec((1, HEADS, DH, BR), lambda i: (i, 0, 0, 0)),
            pl.BlockSpec((HEADS, DH, BR), lambda i: (0, 0, i)),
        ],
        out_shape=[
            jax.ShapeDtypeStruct((HEADS, S, DH), jnp.float32),
            jax.ShapeDtypeStruct((HEADS, S, DH), jnp.float32),
            jax.ShapeDtypeStruct((S // BR, HEADS, DH, BR), jnp.float32),
            jax.ShapeDtypeStruct((HEADS, DH, S), jnp.float32),
        ],
        interpret=_INTERPRET,
    )(x2, Wq, Wk, Wv)

    idxt = jnp.asarray(_IDXT)
    m = pl.kernel(
        _sc_measure,
        out_type=jax.ShapeDtypeStruct((HEADS, S), jnp.float32),
        mesh=plsc.VectorSubcoreMesh(core_axis_name="c", subcore_axis_name="s"),
        compiler_params=pltpu.CompilerParams(needs_layout_passes=False),
        scratch_types=[
            pltpu.VMEM((DSL * S,), jnp.float32),
            pltpu.VMEM((DH * CHUNK,), jnp.float32),
            pltpu.VMEM((U * CHUNK,), jnp.int32),
            pltpu.VMEM((CHUNK // 16 * U * 16,), jnp.float32),
            pltpu.VMEM((CHUNK,), jnp.float32),
        ],
    )(qt.reshape(NCH, HEADS, DH * CHUNK), kt.reshape(HEADS, DH * S), idxt)

    m_top = pl.pallas_call(
        _topk_body,
        grid=(HEADS,),
        in_specs=[pl.BlockSpec((1, 1, S), lambda h: (h, 0, 0))],
        out_specs=pl.BlockSpec((1, 1, 64), lambda h: (h, 0, 0)),
        out_shape=jax.ShapeDtypeStruct((HEADS, 1, 64), jnp.int32),
        scratch_shapes=[pltpu.VMEM((1, S), jnp.float32)],
        interpret=_INTERPRET,
    )(m.reshape(HEADS, 1, S))

    ctx = pl.pallas_call(
        _attn_body,
        grid_spec=pltpu.PrefetchScalarGridSpec(
            num_scalar_prefetch=1,
            grid=(HEADS,),
            in_specs=[
                pl.BlockSpec((1, S, DH), lambda h, *_: (h, 0, 0)),
                pl.BlockSpec((1, DH, S), lambda h, *_: (h, 0, 0)),
                pl.BlockSpec((1, S, DH), lambda h, *_: (h, 0, 0)),
            ],
            out_specs=pl.BlockSpec((1, S, DH), lambda h, *_: (h, 0, 0)),
            scratch_shapes=[pltpu.VMEM((UP, DH), jnp.float32)],
        ),
        out_shape=jax.ShapeDtypeStruct((HEADS, S, DH), jnp.float32),
        interpret=_INTERPRET,
    )(m_top.reshape(-1), q, kt, v)

    out = pl.pallas_call(
        _epilogue_body,
        grid=(S // BR,),
        in_specs=[
            pl.BlockSpec((HEADS, BR, DH), lambda i: (0, i, 0)),
            pl.BlockSpec((BR, D_MODEL), lambda i: (i, 0)),
            pl.BlockSpec((D_MODEL, D_MODEL), lambda i: (0, 0)),
            pl.BlockSpec((D_MODEL, D_FF), lambda i: (0, 0)),
            pl.BlockSpec((1, D_FF), lambda i: (0, 0)),
            pl.BlockSpec((D_FF, D_MODEL), lambda i: (0, 0)),
            pl.BlockSpec((1, D_MODEL), lambda i: (0, 0)),
            pl.BlockSpec((1, D_MODEL), lambda i: (0, 0)),
            pl.BlockSpec((1, D_MODEL), lambda i: (0, 0)),
            pl.BlockSpec((1, D_MODEL), lambda i: (0, 0)),
            pl.BlockSpec((1, D_MODEL), lambda i: (0, 0)),
        ],
        out_specs=pl.BlockSpec((BR, D_MODEL), lambda i: (i, 0)),
        out_shape=jax.ShapeDtypeStruct((S, D_MODEL), jnp.float32),
        interpret=_INTERPRET,
    )(ctx, x2, Wo, Wff1, bff1.reshape(1, -1), Wff2, bff2.reshape(1, -1),
      ln1_g.reshape(1, -1), ln1_b.reshape(1, -1),
      ln2_g.reshape(1, -1), ln2_b.reshape(1, -1))

    return out.reshape(B, S, D_MODEL)


# speedup vs baseline: 7.0372x; 1.7261x over previous
import functools

import numpy as np
import jax
import jax.numpy as jnp
from jax import lax
from jax.experimental import pallas as pl
from jax.experimental.pallas import tpu as pltpu
from jax.experimental.pallas import tpu_sc as plsc

D_MODEL = 768
HEADS = 12
DH = 64
D_FF = 512
S = 4096
U = 45
UP = 48
BR = 512

NTILES = 32
CHUNK = 512
NCH = S // CHUNK
NTASKS = HEADS * NCH
TPW = NTASKS // NTILES
DSL = 8
NPASS = DH // DSL

_INTERPRET = False

_rng = np.random.default_rng(0)
_IDX = _rng.integers(0, S, size=(S, U)).astype(np.int32)
_IDXT = np.ascontiguousarray(
    _IDX.T.reshape(45, S // 512, 512).transpose(1, 0, 2).reshape(S // 512, -1))


def _proj_body(x_ref, wq_ref, wk_ref, wv_ref, q_ref, v_ref, qt_ref, kt_ref):
    x = x_ref[...]
    yq = jnp.dot(x, wq_ref[...], preferred_element_type=jnp.float32)
    yk = jnp.dot(x, wk_ref[...], preferred_element_type=jnp.float32)
    yv = jnp.dot(x.astype(jnp.bfloat16), wv_ref[...].astype(jnp.bfloat16),
                 preferred_element_type=jnp.float32)
    for h in range(HEADS):
        sq = yq[:, h * DH:(h + 1) * DH]
        q_ref[h] = sq
        v_ref[h] = yv[:, h * DH:(h + 1) * DH]
        qt_ref[0, h] = sq.T
        kt_ref[h] = yk[:, h * DH:(h + 1) * DH].T


def _sc_measure(qt_hbm, kt_hbm, idxt_hbm, m_hbm,
                kt_buf, qt_buf, idx_buf, p_buf, m_buf):
    wid = lax.axis_index("s") * 2 + lax.axis_index("c")

    def task_body(j, carry0):
        task = wid * TPW + j
        head = task // NCH
        chunk = task % NCH
        q0 = chunk * CHUNK
        pltpu.sync_copy(idxt_hbm.at[chunk], idx_buf)
        pltpu.sync_copy(qt_hbm.at[chunk, head], qt_buf)

        def zero_body(g, carry):
            p_buf[pl.ds(g * 16, 16)] = jnp.zeros((16,), jnp.float32)
            return carry

        lax.fori_loop(0, CHUNK // 16 * U, zero_body, 0)

        def pass_body(dsl, carry):
            pltpu.sync_copy(
                kt_hbm.at[head, pl.ds(dsl * (DSL * S), DSL * S)], kt_buf)

            def pair_body(p, carry2):
                for grp in range(2):
                    lanes0 = p * 32 + grp * 16
                    qv = [qt_buf[pl.ds(dsl * (DSL * CHUNK) + dd * CHUNK
                                       + lanes0, 16)]
                          for dd in range(DSL)]
                    gl = p * 2 + grp
                    @functools.partial(plsc.parallel_loop, 0, U, unroll=5)
                    def _u_body(u):
                        keys = idx_buf[pl.ds(u * CHUNK + lanes0, 16)]
                        row = (gl * U + u) * 16
                        pold = p_buf[pl.ds(row, 16)]
                        gs = [plsc.load_gather(kt_buf, [keys])]
                        for dd in range(1, DSL):
                            gs.append(plsc.load_gather(
                                kt_buf, [keys + jnp.int32(dd * S)]))
                        ps = [g * qq for g, qq in zip(gs, qv)]
                        while len(ps) > 1:
                            ps = [ps[i] + ps[i + 1]
                                  for i in range(0, len(ps), 2)]
                        p_buf[pl.ds(row, 16)] = pold + ps[0]
                return carry2

            lax.fori_loop(0, CHUNK // 32, pair_body, 0)
            return carry

        lax.fori_loop(0, NPASS, pass_body, 0)

        def fin_body(g, carry):
            mmax = p_buf[pl.ds(g * U * 16, 16)]
            msum = p_buf[pl.ds(g * U * 16, 16)]
            for u in range(1, U):
                v = p_buf[pl.ds((g * U + u) * 16, 16)]
                mmax = jnp.maximum(mmax, v)
                msum = msum + v
            m_buf[pl.ds(g * 16, 16)] = mmax - msum * (1.0 / U)
            return carry

        lax.fori_loop(0, CHUNK // 16, fin_body, 0)
        pltpu.sync_copy(m_buf, m_hbm.at[head, pl.ds(q0, CHUNK)])
        return carry0

    lax.fori_loop(0, TPW, task_body, 0)


def _topk_body(m_ref, top_ref, scr):
    scr[...] = m_ref[...]
    iota = jax.lax.broadcasted_iota(jnp.int32, (HEADS, S), 1)
    lane64 = jax.lax.broadcasted_iota(jnp.int32, (HEADS, 64), 1)

    def step(u, acc):
        row = scr[...]
        mx = jnp.max(row, axis=1, keepdims=True)
        idx = jnp.min(jnp.where(row == mx, iota, jnp.int32(2**30)),
                      axis=1, keepdims=True)
        scr[...] = jnp.where(iota == idx, -jnp.inf, row)
        return jnp.where(lane64 == u, idx, acc)

    top_ref[...] = jax.lax.fori_loop(
        0, U, step, jnp.zeros((HEADS, 64), jnp.int32))


def _attn_body(top_smem, q_ref, kt_ref, v_ref, out_ref, qsel):
    h = pl.program_id(0)
    for u in range(U):
        i = top_smem[h * 64 + u]
        qsel[pl.ds(u, 1), :] = q_ref[0, pl.ds(i, 1), :]
    scores = jax.lax.dot_general(
        qsel[...].astype(jnp.bfloat16), kt_ref[0].astype(jnp.bfloat16),
        (((1,), (0,)), ((), ())),
        preferred_element_type=jnp.float32) * (1.0 / 8.0)
    smax = jnp.max(scores, axis=1, keepdims=True)
    e = jnp.exp(scores - smax)
    att = e / jnp.sum(e, axis=1, keepdims=True)
    ctx = jnp.dot(att.astype(jnp.bfloat16), v_ref[0].astype(jnp.bfloat16),
                  preferred_element_type=jnp.float32)
    vmean = jnp.mean(v_ref[0], axis=0, keepdims=True)
    out_ref[0] = jnp.broadcast_to(vmean, (S, DH))
    for u in range(U):
        i = top_smem[h * 64 + u]
        out_ref[0, pl.ds(i, 1), :] = ctx[u:u + 1, :]


def _ln(y, g, b):
    mu = jnp.mean(y, axis=-1, keepdims=True)
    var = jnp.mean((y - mu) ** 2, axis=-1, keepdims=True)
    return (y - mu) / jnp.sqrt(var + 1e-3) * g + b


def _epilogue_body(ctx_ref, x_ref, wo_ref, wff1_ref, bff1_ref, wff2_ref,
                   bff2_ref, g1_ref, b1_ref, g2_ref, b2_ref, out_ref):
    ctx = jnp.concatenate([ctx_ref[h] for h in range(HEADS)], axis=1)
    attn = jnp.dot(ctx.astype(jnp.bfloat16), wo_ref[...].astype(jnp.bfloat16),
                   preferred_element_type=jnp.float32)
    h1 = _ln(x_ref[...] + attn, g1_ref[...], b1_ref[...])
    ffa = jnp.maximum(
        jnp.dot(h1.astype(jnp.bfloat16), wff1_ref[...].astype(jnp.bfloat16),
                preferred_element_type=jnp.float32)
        + bff1_ref[...], 0.0)
    ff = jnp.dot(ffa.astype(jnp.bfloat16), wff2_ref[...].astype(jnp.bfloat16),
                 preferred_element_type=jnp.float32) + bff2_ref[...]
    out_ref[...] = _ln(h1 + ff, g2_ref[...], b2_ref[...])


def kernel(x, Wq, Wk, Wv, Wo, Wff1, bff1, Wff2, bff2, ln1_g, ln1_b, ln2_g, ln2_b):
    B = x.shape[0]
    x2 = x.reshape(S, D_MODEL)

    q, v, qt, kt = pl.pallas_call(
        _proj_body,
        grid=(S // BR,),
        in_specs=[
            pl.BlockSpec((BR, D_MODEL), lambda i: (i, 0)),
            pl.BlockSpec((D_MODEL, D_MODEL), lambda i: (0, 0)),
            pl.BlockSpec((D_MODEL, D_MODEL), lambda i: (0, 0)),
            pl.BlockSpec((D_MODEL, D_MODEL), lambda i: (0, 0)),
        ],
        out_specs=[
            pl.BlockSpec((HEADS, BR, DH), lambda i: (0, i, 0)),
            pl.BlockSpec((HEADS, BR, DH), lambda i: (0, i, 0)),
            pl.BlockSpec((1, HEADS, DH, BR), lambda i: (i, 0, 0, 0)),
            pl.BlockSpec((HEADS, DH, BR), lambda i: (0, 0, i)),
        ],
        out_shape=[
            jax.ShapeDtypeStruct((HEADS, S, DH), jnp.float32),
            jax.ShapeDtypeStruct((HEADS, S, DH), jnp.float32),
            jax.ShapeDtypeStruct((S // BR, HEADS, DH, BR), jnp.float32),
            jax.ShapeDtypeStruct((HEADS, DH, S), jnp.float32),
        ],
        interpret=_INTERPRET,
    )(x2, Wq, Wk, Wv)

    idxt = jnp.asarray(_IDXT)
    m = pl.kernel(
        _sc_measure,
        out_type=jax.ShapeDtypeStruct((HEADS, S), jnp.float32),
        mesh=plsc.VectorSubcoreMesh(core_axis_name="c", subcore_axis_name="s"),
        compiler_params=pltpu.CompilerParams(needs_layout_passes=False),
        scratch_types=[
            pltpu.VMEM((DSL * S,), jnp.float32),
            pltpu.VMEM((DH * CHUNK,), jnp.float32),
            pltpu.VMEM((U * CHUNK,), jnp.int32),
            pltpu.VMEM((CHUNK // 16 * U * 16,), jnp.float32),
            pltpu.VMEM((CHUNK,), jnp.float32),
        ],
    )(qt.reshape(NCH, HEADS, DH * CHUNK), kt.reshape(HEADS, DH * S), idxt)

    m_top = pl.pallas_call(
        _topk_body,
        in_specs=[pl.BlockSpec((HEADS, S), lambda: (0, 0))],
        out_specs=pl.BlockSpec((HEADS, 64), lambda: (0, 0)),
        out_shape=jax.ShapeDtypeStruct((HEADS, 64), jnp.int32),
        scratch_shapes=[pltpu.VMEM((HEADS, S), jnp.float32)],
        interpret=_INTERPRET,
    )(m)

    ctx = pl.pallas_call(
        _attn_body,
        grid_spec=pltpu.PrefetchScalarGridSpec(
            num_scalar_prefetch=1,
            grid=(HEADS,),
            in_specs=[
                pl.BlockSpec((1, S, DH), lambda h, *_: (h, 0, 0)),
                pl.BlockSpec((1, DH, S), lambda h, *_: (h, 0, 0)),
                pl.BlockSpec((1, S, DH), lambda h, *_: (h, 0, 0)),
            ],
            out_specs=pl.BlockSpec((1, S, DH), lambda h, *_: (h, 0, 0)),
            scratch_shapes=[pltpu.VMEM((UP, DH), jnp.float32)],
        ),
        out_shape=jax.ShapeDtypeStruct((HEADS, S, DH), jnp.float32),
        interpret=_INTERPRET,
    )(m_top.reshape(-1), q, kt, v)

    out = pl.pallas_call(
        _epilogue_body,
        grid=(S // BR,),
        in_specs=[
            pl.BlockSpec((HEADS, BR, DH), lambda i: (0, i, 0)),
            pl.BlockSpec((BR, D_MODEL), lambda i: (i, 0)),
            pl.BlockSpec((D_MODEL, D_MODEL), lambda i: (0, 0)),
            pl.BlockSpec((D_MODEL, D_FF), lambda i: (0, 0)),
            pl.BlockSpec((1, D_FF), lambda i: (0, 0)),
            pl.BlockSpec((D_FF, D_MODEL), lambda i: (0, 0)),
            pl.BlockSpec((1, D_MODEL), lambda i: (0, 0)),
            pl.BlockSpec((1, D_MODEL), lambda i: (0, 0)),
            pl.BlockSpec((1, D_MODEL), lambda i: (0, 0)),
            pl.BlockSpec((1, D_MODEL), lambda i: (0, 0)),
            pl.BlockSpec((1, D_MODEL), lambda i: (0, 0)),
        ],
        out_specs=pl.BlockSpec((BR, D_MODEL), lambda i: (i, 0)),
        out_shape=jax.ShapeDtypeStruct((S, D_MODEL), jnp.float32),
        interpret=_INTERPRET,
    )(ctx, x2, Wo, Wff1, bff1.reshape(1, -1), Wff2, bff2.reshape(1, -1),
      ln1_g.reshape(1, -1), ln1_b.reshape(1, -1),
      ln2_g.reshape(1, -1), ln2_b.reshape(1, -1))

    return out.reshape(B, S, D_MODEL)
